# baseline (speedup 1.0000x reference)
import functools

import jax
import jax.numpy as jnp
from jax import lax
from jax.experimental import pallas as pl
from jax.experimental.pallas import tpu as pltpu
from jax.experimental.pallas import tpu_sc as plsc

ROWS = 4096
COLS = 100000
NSEL = 256
SUBL = 8
NBAND = ROWS // SUBL
W = 128
WSHIFT = 7

NC = 2
NS = 16
NW = NC * NS

ROWS_PER_W = ROWS // NW
BANDS_PER_W = NBAND // NW
ELEMS_PER_W = ROWS_PER_W * NSEL
DEPTH = 4


def _gather_body(logits_hbm, map_hbm, out_hbm, m_v, bidx_v, slabs, dat_v, sems):
    c_id = lax.axis_index("c")
    s_id = lax.axis_index("s")
    wid = s_id * NC + c_id
    base_band = wid * BANDS_PER_W

    pltpu.sync_copy(map_hbm, m_v)

    iota16 = lax.iota(jnp.int32, 16)
    iota_rows = iota16 * NSEL
    bidx_v[pl.ds(0, 16)] = iota16 + base_band

    view3 = logits_hbm.reshape(NBAND, SUBL, COLS)

    def col_of(j):
        mvec = m_v[pl.ds((j >> 4) << 4, 16)]
        return jnp.sum(jnp.where(iota16 == (j & 15), mvec, 0))

    def fire(j):
        mj = col_of(j)
        col0 = pl.multiple_of((mj >> WSHIFT) << WSHIFT, W)
        slot = j % DEPTH
        pltpu.async_copy(
            view3.at[bidx_v, pl.ds(0, SUBL), pl.ds(col0, W)],
            slabs.at[slot],
            sems.at[slot],
        )

    def wait(j):
        slot = j % DEPTH
        pltpu.make_async_copy(
            view3.at[bidx_v, pl.ds(0, SUBL), pl.ds(0, W)],
            slabs.at[slot],
            sems.at[slot],
        ).wait()

    def prologue(j, c):
        fire(j)
        return c

    lax.fori_loop(0, DEPTH - 1, prologue, 0)

    def do_col(j, c):
        @pl.when(j + DEPTH - 1 < NSEL)
        def _():
            fire(j + DEPTH - 1)

        wait(j)
        slot = j % DEPTH
        mj = col_of(j)
        lane = jnp.full((16,), mj & (W - 1), dtype=jnp.int32)

        def rowvec(v, c2):
            i0 = v * 16
            i = iota16 + i0
            vals = plsc.load_gather(slabs.at[slot], [i >> 3, i & 7, lane])
            plsc.store_scatter(dat_v, [iota_rows + (i0 * NSEL + j)], vals)
            return c2

        lax.fori_loop(0, ROWS_PER_W // 16, rowvec, 0)
        return c

    lax.fori_loop(0, NSEL, do_col, 0)

    pltpu.sync_copy(dat_v, out_hbm.at[pl.ds(wid * ELEMS_PER_W, ELEMS_PER_W)])


_sc_gather = pl.kernel(
    _gather_body,
    out_type=jax.ShapeDtypeStruct((ROWS * NSEL,), jnp.float32),
    mesh=plsc.VectorSubcoreMesh(
        core_axis_name="c", subcore_axis_name="s", num_cores=NC, num_subcores=NS
    ),
    compiler_params=pltpu.CompilerParams(needs_layout_passes=False),
    scratch_types=[
        pltpu.VMEM((NSEL,), jnp.int32),
        pltpu.VMEM((16,), jnp.int32),
        pltpu.VMEM((DEPTH, BANDS_PER_W, SUBL, W), jnp.float32),
        pltpu.VMEM((ELEMS_PER_W,), jnp.float32),
        pltpu.SemaphoreType.DMA((DEPTH,)),
    ],
)


@jax.jit
def kernel(logits, mapping_sequence):
    out = _sc_gather(logits, mapping_sequence.astype(jnp.int32))
    return out.reshape(ROWS, NSEL)
